# Initial kernel scaffold; baseline (speedup 1.0000x reference)
import jax
import jax.numpy as jnp
from jax.experimental import pallas as pl


def kernel(x, edge_index, batch_index, w1_0, w2_0, w1_1, w2_1, w1_2, w2_2, out_w, out_b):
    raise NotImplementedError("write your pallas kernel here")


# speedup vs baseline: 6.8244x; 6.8244x over previous
import functools

import jax
import jax.numpy as jnp
from jax import lax
from jax.experimental import pallas as pl
from jax.experimental.pallas import tpu as pltpu
from jax.experimental.pallas import tpu_sc as plsc

N_NODES = 10000
N_EDGES = 320000
FEAT = 128
N_GRAPHS = 64

NC = 2
NS = 16
CHUNK = 128
CH_PER_TILE = 79
E_PAD = NC * NS * CH_PER_TILE * CHUNK
N_PAD = 10112
STRIPE = N_PAD // NS


def _sc_aggregate(h, src_r, dst_r, zeros):
    mesh = plsc.VectorSubcoreMesh(core_axis_name="c", subcore_axis_name="s")

    @functools.partial(
        pl.kernel,
        mesh=mesh,
        out_type=jax.ShapeDtypeStruct((NC, N_PAD, FEAT), jnp.float32),
        scratch_types=[
            pltpu.VMEM((CH_PER_TILE, CHUNK), jnp.int32),
            pltpu.VMEM((CH_PER_TILE, CHUNK), jnp.int32),
            pltpu.VMEM((CHUNK, FEAT), jnp.float32),
            pltpu.VMEM_SHARED((N_PAD, FEAT), jnp.float32),
            pltpu.SemaphoreType.DMA,
        ],
    )
    def agg_kernel(h_hbm, src_hbm, dst_hbm, zeros_hbm, out_hbm,
                   src_v, dst_v, rows_v, agg_sh, sem):
        cid = lax.axis_index("c")
        sid = lax.axis_index("s")
        pltpu.sync_copy(src_hbm.at[cid, sid], src_v)
        pltpu.sync_copy(dst_hbm.at[cid, sid], dst_v)
        pltpu.sync_copy(zeros_hbm, agg_sh.at[pl.ds(sid * STRIPE, STRIPE)])
        plsc.subcore_barrier()

        def body(j, carry):
            pltpu.async_copy(h_hbm.at[src_v.at[j]], rows_v, sem).wait()
            pltpu.sync_copy(rows_v, agg_sh.at[dst_v.at[j]], add=True)
            return carry

        lax.fori_loop(0, CH_PER_TILE, body, 0)
        plsc.subcore_barrier()
        pltpu.sync_copy(agg_sh.at[pl.ds(sid * STRIPE, STRIPE)],
                        out_hbm.at[cid, pl.ds(sid * STRIPE, STRIPE)])

    return agg_kernel(h, src_r, dst_r, zeros)


def _mlp_body(h_ref, a0_ref, a1_ref, w1_ref, w2_ref, o_ref):
    z = h_ref[...] + a0_ref[0] + a1_ref[0]
    z = jnp.maximum(jnp.dot(z, w1_ref[...], preferred_element_type=jnp.float32, precision=jax.lax.Precision.HIGHEST), 0.0)
    z = jnp.maximum(jnp.dot(z, w2_ref[...], preferred_element_type=jnp.float32, precision=jax.lax.Precision.HIGHEST), 0.0)
    o_ref[...] = z


def _mlp(h, agg2, w1, w2):
    blk = 1000
    return pl.pallas_call(
        _mlp_body,
        grid=(N_NODES // blk,),
        in_specs=[
            pl.BlockSpec((blk, FEAT), lambda i: (i, 0)),
            pl.BlockSpec((1, blk, FEAT), lambda i: (0, i, 0)),
            pl.BlockSpec((1, blk, FEAT), lambda i: (1, i, 0)),
            pl.BlockSpec((FEAT, FEAT), lambda i: (0, 0)),
            pl.BlockSpec((FEAT, FEAT), lambda i: (0, 0)),
        ],
        out_specs=pl.BlockSpec((blk, FEAT), lambda i: (i, 0)),
        out_shape=jax.ShapeDtypeStruct((N_NODES, FEAT), jnp.float32),
    )(h, agg2, agg2, w1, w2)


def _pool_body(h_ref, b_ref, ow_ref, ob_ref, out_ref, off_sm, maxs_vm):
    batv = b_ref[...]

    def off_body(g, carry):
        off_sm[g] = jnp.sum((batv < g).astype(jnp.int32))
        return carry

    lax.fori_loop(0, N_GRAPHS + 1, off_body, 0)

    gid = lax.broadcasted_iota(jnp.int32, (N_GRAPHS, N_NODES), 0)
    onehot = (gid == batv).astype(jnp.float32)
    sums = jnp.dot(onehot, h_ref[...], preferred_element_type=jnp.float32, precision=jax.lax.Precision.HIGHEST)
    counts = jnp.sum(onehot, axis=1, keepdims=True)

    RB = 256

    def g_body(g, carry):
        start = off_sm[g]
        end = off_sm[g + 1]
        nb = (end - start + RB - 1) // RB

        def k_body(k, acc):
            st = start + k * RB
            stc = jnp.minimum(st, N_NODES - RB)
            rows = h_ref[pl.ds(stc, RB), :]
            ridx = stc + lax.broadcasted_iota(jnp.int32, (RB, 1), 0)
            m = (ridx >= start) & (ridx < end)
            blkmax = jnp.max(jnp.where(m, rows, -jnp.inf), axis=0, keepdims=True)
            return jnp.maximum(acc, blkmax)

        acc = lax.fori_loop(0, nb, k_body,
                            jnp.full((1, FEAT), -jnp.inf, jnp.float32))
        maxs_vm[pl.ds(g, 1), :] = acc
        return carry

    lax.fori_loop(0, N_GRAPHS, g_body, 0)

    maxs = jnp.where(counts > 0, maxs_vm[...], 0.0)
    means = sums / jnp.maximum(counts, 1.0)
    pooled = jnp.concatenate([maxs, means, sums], axis=1)
    out_ref[...] = (jnp.dot(pooled, ow_ref[...], preferred_element_type=jnp.float32, precision=jax.lax.Precision.HIGHEST)
                    + ob_ref[...])


def _pool(h, batch2d, out_w, out_b2d):
    return pl.pallas_call(
        _pool_body,
        in_specs=[
            pl.BlockSpec((N_NODES, FEAT), lambda: (0, 0)),
            pl.BlockSpec((1, N_NODES), lambda: (0, 0)),
            pl.BlockSpec((3 * FEAT, FEAT), lambda: (0, 0)),
            pl.BlockSpec((1, FEAT), lambda: (0, 0)),
        ],
        out_specs=pl.BlockSpec((N_GRAPHS, FEAT), lambda: (0, 0)),
        out_shape=jax.ShapeDtypeStruct((N_GRAPHS, FEAT), jnp.float32),
        scratch_shapes=[
            pltpu.SMEM((N_GRAPHS + 1,), jnp.int32),
            pltpu.VMEM((N_GRAPHS, FEAT), jnp.float32),
        ],
    )(h, batch2d, out_w, out_b2d)


def kernel(x, edge_index, batch_index, w1_0, w2_0, w1_1, w2_1, w1_2, w2_2,
           out_w, out_b):
    src, dst = edge_index[0], edge_index[1]
    pad_n = E_PAD - N_EDGES
    pad_i = jnp.arange(pad_n, dtype=jnp.int32)
    src_p = jnp.concatenate([src, pad_i % N_NODES])
    dst_p = jnp.concatenate([dst, N_NODES + (pad_i % (N_PAD - N_NODES))])
    src_r = src_p.reshape(NC, NS, CH_PER_TILE, CHUNK)
    dst_r = dst_p.reshape(NC, NS, CH_PER_TILE, CHUNK)
    zeros = jnp.zeros((STRIPE, FEAT), jnp.float32)
    batch2d = batch_index.reshape(1, N_NODES)
    out_b2d = out_b.reshape(1, FEAT)

    h = x
    for (w1, w2) in ((w1_0, w2_0), (w1_1, w2_1), (w1_2, w2_2)):
        agg2 = _sc_aggregate(h, src_r, dst_r, zeros)
        h = _mlp(h, agg2, w1, w2)
    return _pool(h, batch2d, out_w, out_b2d)


# speedup vs baseline: 8.4857x; 1.2434x over previous
import functools

import jax
import jax.numpy as jnp
from jax import lax
from jax.experimental import pallas as pl
from jax.experimental.pallas import tpu as pltpu
from jax.experimental.pallas import tpu_sc as plsc

N_NODES = 10000
N_EDGES = 320000
FEAT = 128
N_GRAPHS = 64

NC = 2
NS = 16
CHUNK = 128
CH_PER_TILE = 79
E_PAD = NC * NS * CH_PER_TILE * CHUNK
PHASE_CH = 40
N_PHASES = 2
N_PAD = 10112
STRIPE = N_PAD // NS


def _sc_aggregate(h, src_r, dst_r, zeros):
    mesh = plsc.VectorSubcoreMesh(core_axis_name="c", subcore_axis_name="s")

    @functools.partial(
        pl.kernel,
        mesh=mesh,
        out_type=jax.ShapeDtypeStruct((NC, N_PAD, FEAT), jnp.float32),
        scratch_types=[
            pltpu.VMEM((PHASE_CH, CHUNK), jnp.int32),
            pltpu.VMEM((PHASE_CH, CHUNK), jnp.int32),
            pltpu.VMEM((2, CHUNK, FEAT), jnp.float32),
            pltpu.VMEM_SHARED((N_PAD, FEAT), jnp.float32),
            pltpu.SemaphoreType.DMA,
            pltpu.SemaphoreType.DMA,
        ],
    )
    def agg_kernel(h_hbm, src_hbm, dst_hbm, zeros_hbm, out_hbm,
                   src_v, dst_v, rows_v, agg_sh, sem_g, sem_s):
        cid = lax.axis_index("c")
        sid = lax.axis_index("s")
        pltpu.sync_copy(zeros_hbm, agg_sh.at[pl.ds(sid * STRIPE, STRIPE)])
        plsc.subcore_barrier()

        for p in range(N_PHASES):
            n_ch = min(PHASE_CH, CH_PER_TILE - p * PHASE_CH)
            pltpu.sync_copy(src_hbm.at[cid, sid, pl.ds(p * PHASE_CH, n_ch)],
                            src_v.at[pl.ds(0, n_ch)])
            pltpu.sync_copy(dst_hbm.at[cid, sid, pl.ds(p * PHASE_CH, n_ch)],
                            dst_v.at[pl.ds(0, n_ch)])

            pltpu.async_copy(h_hbm.at[src_v.at[0]], rows_v.at[0], sem_g)

            def body(j, carry):
                b = lax.rem(j, 2)
                pltpu.make_async_copy(h_hbm.at[src_v.at[j]], rows_v.at[b],
                                      sem_g).wait()

                @pl.when(j > 0)
                def _():
                    pltpu.make_async_copy(rows_v.at[1 - b],
                                          agg_sh.at[dst_v.at[j - 1]],
                                          sem_s).wait()

                @pl.when(j < n_ch - 1)
                def _():
                    pltpu.async_copy(h_hbm.at[src_v.at[j + 1]],
                                     rows_v.at[1 - b], sem_g)

                pltpu.async_copy(rows_v.at[b], agg_sh.at[dst_v.at[j]], sem_s,
                                 add=True)
                return carry

            lax.fori_loop(0, n_ch, body, 0)
            pltpu.make_async_copy(rows_v.at[(n_ch - 1) % 2],
                                  agg_sh.at[dst_v.at[n_ch - 1]],
                                  sem_s).wait()
        plsc.subcore_barrier()
        pltpu.sync_copy(agg_sh.at[pl.ds(sid * STRIPE, STRIPE)],
                        out_hbm.at[cid, pl.ds(sid * STRIPE, STRIPE)])

    return agg_kernel(h, src_r, dst_r, zeros)


def _mlp_body(h_ref, a0_ref, a1_ref, w1_ref, w2_ref, o_ref):
    z = h_ref[...] + a0_ref[0] + a1_ref[0]
    z = jnp.maximum(jnp.dot(z, w1_ref[...], preferred_element_type=jnp.float32, precision=jax.lax.Precision.HIGHEST), 0.0)
    z = jnp.maximum(jnp.dot(z, w2_ref[...], preferred_element_type=jnp.float32, precision=jax.lax.Precision.HIGHEST), 0.0)
    o_ref[...] = z


def _mlp(h, agg2, w1, w2):
    blk = 1000
    return pl.pallas_call(
        _mlp_body,
        grid=(N_NODES // blk,),
        in_specs=[
            pl.BlockSpec((blk, FEAT), lambda i: (i, 0)),
            pl.BlockSpec((1, blk, FEAT), lambda i: (0, i, 0)),
            pl.BlockSpec((1, blk, FEAT), lambda i: (1, i, 0)),
            pl.BlockSpec((FEAT, FEAT), lambda i: (0, 0)),
            pl.BlockSpec((FEAT, FEAT), lambda i: (0, 0)),
        ],
        out_specs=pl.BlockSpec((blk, FEAT), lambda i: (i, 0)),
        out_shape=jax.ShapeDtypeStruct((N_NODES, FEAT), jnp.float32),
    )(h, agg2, agg2, w1, w2)


def _pool_body(h_ref, b_ref, ow_ref, ob_ref, out_ref, off_sm, maxs_vm):
    batv = b_ref[...]

    def off_body(g, carry):
        off_sm[g] = jnp.sum((batv < g).astype(jnp.int32))
        return carry

    lax.fori_loop(0, N_GRAPHS + 1, off_body, 0)

    gid = lax.broadcasted_iota(jnp.int32, (N_GRAPHS, N_NODES), 0)
    onehot = (gid == batv).astype(jnp.float32)
    sums = jnp.dot(onehot, h_ref[...], preferred_element_type=jnp.float32, precision=jax.lax.Precision.HIGHEST)
    counts = jnp.sum(onehot, axis=1, keepdims=True)

    RB = 256

    def g_body(g, carry):
        start = off_sm[g]
        end = off_sm[g + 1]
        nb = (end - start + RB - 1) // RB

        def k_body(k, acc):
            st = start + k * RB
            stc = jnp.minimum(st, N_NODES - RB)
            rows = h_ref[pl.ds(stc, RB), :]
            ridx = stc + lax.broadcasted_iota(jnp.int32, (RB, 1), 0)
            m = (ridx >= start) & (ridx < end)
            blkmax = jnp.max(jnp.where(m, rows, -jnp.inf), axis=0, keepdims=True)
            return jnp.maximum(acc, blkmax)

        acc = lax.fori_loop(0, nb, k_body,
                            jnp.full((1, FEAT), -jnp.inf, jnp.float32))
        maxs_vm[pl.ds(g, 1), :] = acc
        return carry

    lax.fori_loop(0, N_GRAPHS, g_body, 0)

    maxs = jnp.where(counts > 0, maxs_vm[...], 0.0)
    means = sums / jnp.maximum(counts, 1.0)
    pooled = jnp.concatenate([maxs, means, sums], axis=1)
    out_ref[...] = (jnp.dot(pooled, ow_ref[...], preferred_element_type=jnp.float32, precision=jax.lax.Precision.HIGHEST)
                    + ob_ref[...])


def _pool(h, batch2d, out_w, out_b2d):
    return pl.pallas_call(
        _pool_body,
        in_specs=[
            pl.BlockSpec((N_NODES, FEAT), lambda: (0, 0)),
            pl.BlockSpec((1, N_NODES), lambda: (0, 0)),
            pl.BlockSpec((3 * FEAT, FEAT), lambda: (0, 0)),
            pl.BlockSpec((1, FEAT), lambda: (0, 0)),
        ],
        out_specs=pl.BlockSpec((N_GRAPHS, FEAT), lambda: (0, 0)),
        out_shape=jax.ShapeDtypeStruct((N_GRAPHS, FEAT), jnp.float32),
        scratch_shapes=[
            pltpu.SMEM((N_GRAPHS + 1,), jnp.int32),
            pltpu.VMEM((N_GRAPHS, FEAT), jnp.float32),
        ],
    )(h, batch2d, out_w, out_b2d)


def kernel(x, edge_index, batch_index, w1_0, w2_0, w1_1, w2_1, w1_2, w2_2,
           out_w, out_b):
    src, dst = edge_index[0], edge_index[1]
    pad_n = E_PAD - N_EDGES
    pad_i = jnp.arange(pad_n, dtype=jnp.int32)
    src_p = jnp.concatenate([src, pad_i % N_NODES])
    dst_p = jnp.concatenate([dst, N_NODES + (pad_i % (N_PAD - N_NODES))])
    src_r = src_p.reshape(NC, NS, CH_PER_TILE, CHUNK)
    dst_r = dst_p.reshape(NC, NS, CH_PER_TILE, CHUNK)
    zeros = jnp.zeros((STRIPE, FEAT), jnp.float32)
    batch2d = batch_index.reshape(1, N_NODES)
    out_b2d = out_b.reshape(1, FEAT)

    h = x
    for (w1, w2) in ((w1_0, w2_0), (w1_1, w2_1), (w1_2, w2_2)):
        agg2 = _sc_aggregate(h, src_r, dst_r, zeros)
        h = _mlp(h, agg2, w1, w2)
    return _pool(h, batch2d, out_w, out_b2d)


# speedup vs baseline: 9.2372x; 1.0886x over previous
import functools

import jax
import jax.numpy as jnp
from jax import lax
from jax.experimental import pallas as pl
from jax.experimental.pallas import tpu as pltpu
from jax.experimental.pallas import tpu_sc as plsc

N_NODES = 10000
N_EDGES = 320000
FEAT = 128
N_GRAPHS = 64

NC = 2
NS = 16
CHUNK = 128
CH_PER_TILE = 79
E_PAD = NC * NS * CH_PER_TILE * CHUNK
PHASE_CH = 40
N_PHASES = 2
N_PAD = 10112
STRIPE = N_PAD // NS


def _sc_aggregate(h, src_r, dst_r, zeros):
    mesh = plsc.VectorSubcoreMesh(core_axis_name="c", subcore_axis_name="s")

    @functools.partial(
        pl.kernel,
        mesh=mesh,
        out_type=jax.ShapeDtypeStruct((NC, N_PAD, FEAT), jnp.float32),
        scratch_types=[
            pltpu.VMEM((PHASE_CH, CHUNK), jnp.int32),
            pltpu.VMEM((PHASE_CH, CHUNK), jnp.int32),
            pltpu.VMEM((2, CHUNK, FEAT), jnp.float32),
            pltpu.VMEM_SHARED((N_PAD, FEAT), jnp.float32),
            pltpu.SemaphoreType.DMA,
            pltpu.SemaphoreType.DMA,
        ],
    )
    def agg_kernel(h_hbm, src_hbm, dst_hbm, zeros_hbm, out_hbm,
                   src_v, dst_v, rows_v, agg_sh, sem_g, sem_s):
        cid = lax.axis_index("c")
        sid = lax.axis_index("s")
        pltpu.sync_copy(zeros_hbm, agg_sh.at[pl.ds(sid * STRIPE, STRIPE)])
        plsc.subcore_barrier()

        for p in range(N_PHASES):
            n_ch = min(PHASE_CH, CH_PER_TILE - p * PHASE_CH)
            pltpu.sync_copy(src_hbm.at[cid, sid, pl.ds(p * PHASE_CH, n_ch)],
                            src_v.at[pl.ds(0, n_ch)])
            pltpu.sync_copy(dst_hbm.at[cid, sid, pl.ds(p * PHASE_CH, n_ch)],
                            dst_v.at[pl.ds(0, n_ch)])

            pltpu.async_copy(h_hbm.at[src_v.at[0]], rows_v.at[0], sem_g)

            def body(j, carry):
                b = lax.rem(j, 2)
                pltpu.make_async_copy(h_hbm.at[src_v.at[j]], rows_v.at[b],
                                      sem_g).wait()

                @pl.when(j > 0)
                def _():
                    pltpu.make_async_copy(rows_v.at[1 - b],
                                          agg_sh.at[dst_v.at[j - 1]],
                                          sem_s).wait()

                @pl.when(j < n_ch - 1)
                def _():
                    pltpu.async_copy(h_hbm.at[src_v.at[j + 1]],
                                     rows_v.at[1 - b], sem_g)

                pltpu.async_copy(rows_v.at[b], agg_sh.at[dst_v.at[j]], sem_s,
                                 add=True)
                return carry

            lax.fori_loop(0, n_ch, body, 0)
            pltpu.make_async_copy(rows_v.at[(n_ch - 1) % 2],
                                  agg_sh.at[dst_v.at[n_ch - 1]],
                                  sem_s).wait()
        plsc.subcore_barrier()
        pltpu.sync_copy(agg_sh.at[pl.ds(sid * STRIPE, STRIPE)],
                        out_hbm.at[cid, pl.ds(sid * STRIPE, STRIPE)])

    return agg_kernel(h, src_r, dst_r, zeros)


def _dot3(a, b):
    ah = a.astype(jnp.bfloat16)
    al = (a - ah.astype(jnp.float32)).astype(jnp.bfloat16)
    bh = b.astype(jnp.bfloat16)
    bl = (b - bh.astype(jnp.float32)).astype(jnp.bfloat16)

    def d(x, y):
        return jax.lax.dot_general(x, y, (((1,), (0,)), ((), ())),
                                   preferred_element_type=jnp.float32)

    return d(ah, bh) + (d(ah, bl) + d(al, bh))


def _mlp_body(h_ref, a0_ref, a1_ref, w1_ref, w2_ref, o_ref):
    z = h_ref[...] + a0_ref[0] + a1_ref[0]
    z = jnp.maximum(_dot3(z, w1_ref[...]), 0.0)
    z = jnp.maximum(_dot3(z, w2_ref[...]), 0.0)
    o_ref[...] = z


def _mlp(h, agg2, w1, w2):
    blk = 1000
    return pl.pallas_call(
        _mlp_body,
        grid=(N_NODES // blk,),
        in_specs=[
            pl.BlockSpec((blk, FEAT), lambda i: (i, 0)),
            pl.BlockSpec((1, blk, FEAT), lambda i: (0, i, 0)),
            pl.BlockSpec((1, blk, FEAT), lambda i: (1, i, 0)),
            pl.BlockSpec((FEAT, FEAT), lambda i: (0, 0)),
            pl.BlockSpec((FEAT, FEAT), lambda i: (0, 0)),
        ],
        out_specs=pl.BlockSpec((blk, FEAT), lambda i: (i, 0)),
        out_shape=jax.ShapeDtypeStruct((N_NODES, FEAT), jnp.float32),
    )(h, agg2, agg2, w1, w2)


def _pool_body(h_ref, b_ref, ow_ref, ob_ref, out_ref, off_sm, maxs_vm):
    batv = b_ref[...]

    def off_body(g, carry):
        off_sm[g] = jnp.sum((batv < g).astype(jnp.int32))
        return carry

    lax.fori_loop(0, N_GRAPHS + 1, off_body, 0)

    gid = lax.broadcasted_iota(jnp.int32, (N_GRAPHS, N_NODES), 0)
    onehot = (gid == batv).astype(jnp.float32)
    sums = _dot3(onehot, h_ref[...])
    counts = jnp.sum(onehot, axis=1, keepdims=True)

    RB = 256

    def g_body(g, carry):
        start = off_sm[g]
        end = off_sm[g + 1]
        nb = (end - start + RB - 1) // RB

        def k_body(k, acc):
            st = start + k * RB
            stc = jnp.minimum(st, N_NODES - RB)
            rows = h_ref[pl.ds(stc, RB), :]
            ridx = stc + lax.broadcasted_iota(jnp.int32, (RB, 1), 0)
            m = (ridx >= start) & (ridx < end)
            blkmax = jnp.max(jnp.where(m, rows, -jnp.inf), axis=0, keepdims=True)
            return jnp.maximum(acc, blkmax)

        acc = lax.fori_loop(0, nb, k_body,
                            jnp.full((1, FEAT), -jnp.inf, jnp.float32))
        maxs_vm[pl.ds(g, 1), :] = acc
        return carry

    lax.fori_loop(0, N_GRAPHS, g_body, 0)

    maxs = jnp.where(counts > 0, maxs_vm[...], 0.0)
    means = sums / jnp.maximum(counts, 1.0)
    pooled = jnp.concatenate([maxs, means, sums], axis=1)
    out_ref[...] = _dot3(pooled, ow_ref[...]) + ob_ref[...]


def _pool(h, batch2d, out_w, out_b2d):
    return pl.pallas_call(
        _pool_body,
        in_specs=[
            pl.BlockSpec((N_NODES, FEAT), lambda: (0, 0)),
            pl.BlockSpec((1, N_NODES), lambda: (0, 0)),
            pl.BlockSpec((3 * FEAT, FEAT), lambda: (0, 0)),
            pl.BlockSpec((1, FEAT), lambda: (0, 0)),
        ],
        out_specs=pl.BlockSpec((N_GRAPHS, FEAT), lambda: (0, 0)),
        out_shape=jax.ShapeDtypeStruct((N_GRAPHS, FEAT), jnp.float32),
        scratch_shapes=[
            pltpu.SMEM((N_GRAPHS + 1,), jnp.int32),
            pltpu.VMEM((N_GRAPHS, FEAT), jnp.float32),
        ],
    )(h, batch2d, out_w, out_b2d)


def kernel(x, edge_index, batch_index, w1_0, w2_0, w1_1, w2_1, w1_2, w2_2,
           out_w, out_b):
    src, dst = edge_index[0], edge_index[1]
    pad_n = E_PAD - N_EDGES
    pad_i = jnp.arange(pad_n, dtype=jnp.int32)
    src_p = jnp.concatenate([src, pad_i % N_NODES])
    dst_p = jnp.concatenate([dst, N_NODES + (pad_i % (N_PAD - N_NODES))])
    src_r = src_p.reshape(NC, NS, CH_PER_TILE, CHUNK)
    dst_r = dst_p.reshape(NC, NS, CH_PER_TILE, CHUNK)
    zeros = jnp.zeros((STRIPE, FEAT), jnp.float32)
    batch2d = batch_index.reshape(1, N_NODES)
    out_b2d = out_b.reshape(1, FEAT)

    h = x
    for (w1, w2) in ((w1_0, w2_0), (w1_1, w2_1), (w1_2, w2_2)):
        agg2 = _sc_aggregate(h, src_r, dst_r, zeros)
        h = _mlp(h, agg2, w1, w2)
    return _pool(h, batch2d, out_w, out_b2d)


# speedup vs baseline: 11.3643x; 1.2303x over previous
import functools

import jax
import jax.numpy as jnp
from jax import lax
from jax.experimental import pallas as pl
from jax.experimental.pallas import tpu as pltpu
from jax.experimental.pallas import tpu_sc as plsc

N_NODES = 10000
N_EDGES = 320000
FEAT = 128
N_GRAPHS = 64

NC = 2
NS = 16
CHUNK = 96
CH_PER_TILE = 106
E_PAD = NC * NS * CH_PER_TILE * CHUNK
PHASE_CH = 40
N_PHASES = 3
NBUF = 3
N_PAD = 10112
STRIPE = N_PAD // NS


def _sc_aggregate(h, src_r, dst_r, zeros):
    mesh = plsc.VectorSubcoreMesh(core_axis_name="c", subcore_axis_name="s")

    @functools.partial(
        pl.kernel,
        mesh=mesh,
        out_type=jax.ShapeDtypeStruct((NC, N_PAD, FEAT), jnp.float32),
        scratch_types=[
            pltpu.VMEM((PHASE_CH, CHUNK), jnp.int32),
            pltpu.VMEM((PHASE_CH, CHUNK), jnp.int32),
            pltpu.VMEM((NBUF, CHUNK, FEAT), jnp.float32),
            pltpu.VMEM_SHARED((N_PAD, FEAT), jnp.float32),
            pltpu.SemaphoreType.DMA,
            pltpu.SemaphoreType.DMA,
        ],
    )
    def agg_kernel(h_hbm, src_hbm, dst_hbm, zeros_hbm, out_hbm,
                   src_v, dst_v, rows_v, agg_sh, sem_g, sem_s):
        cid = lax.axis_index("c")
        sid = lax.axis_index("s")
        pltpu.sync_copy(zeros_hbm, agg_sh.at[pl.ds(sid * STRIPE, STRIPE)])
        plsc.subcore_barrier()

        for p in range(N_PHASES):
            n_ch = min(PHASE_CH, CH_PER_TILE - p * PHASE_CH)
            pltpu.sync_copy(src_hbm.at[cid, sid, pl.ds(p * PHASE_CH, n_ch)],
                            src_v.at[pl.ds(0, n_ch)])
            pltpu.sync_copy(dst_hbm.at[cid, sid, pl.ds(p * PHASE_CH, n_ch)],
                            dst_v.at[pl.ds(0, n_ch)])

            pltpu.async_copy(h_hbm.at[src_v.at[0]], rows_v.at[0], sem_g)
            if n_ch > 1:
                pltpu.async_copy(h_hbm.at[src_v.at[1]], rows_v.at[1], sem_g)

            def body(j, carry):
                b = lax.rem(j, NBUF)
                pltpu.make_async_copy(h_hbm.at[src_v.at[j]], rows_v.at[b],
                                      sem_g).wait()

                @pl.when(j > 0)
                def _():
                    pltpu.make_async_copy(rows_v.at[lax.rem(j - 1, NBUF)],
                                          agg_sh.at[dst_v.at[j - 1]],
                                          sem_s).wait()

                @pl.when(j + 2 < n_ch)
                def _():
                    pltpu.async_copy(h_hbm.at[src_v.at[j + 2]],
                                     rows_v.at[lax.rem(j + 2, NBUF)], sem_g)

                pltpu.async_copy(rows_v.at[b], agg_sh.at[dst_v.at[j]], sem_s,
                                 add=True)
                return carry

            lax.fori_loop(0, n_ch, body, 0)
            pltpu.make_async_copy(rows_v.at[(n_ch - 1) % NBUF],
                                  agg_sh.at[dst_v.at[n_ch - 1]],
                                  sem_s).wait()
        plsc.subcore_barrier()
        pltpu.sync_copy(agg_sh.at[pl.ds(sid * STRIPE, STRIPE)],
                        out_hbm.at[cid, pl.ds(sid * STRIPE, STRIPE)])

    return agg_kernel(h, src_r, dst_r, zeros)


def _dot3(a, b):
    ah = a.astype(jnp.bfloat16)
    al = (a - ah.astype(jnp.float32)).astype(jnp.bfloat16)
    bh = b.astype(jnp.bfloat16)
    bl = (b - bh.astype(jnp.float32)).astype(jnp.bfloat16)

    def d(x, y):
        return jax.lax.dot_general(x, y, (((1,), (0,)), ((), ())),
                                   preferred_element_type=jnp.float32)

    return d(ah, bh) + (d(ah, bl) + d(al, bh))


def _mlp_body(h_ref, a0_ref, a1_ref, w1_ref, w2_ref, o_ref):
    z = h_ref[...] + a0_ref[0] + a1_ref[0]
    z = jnp.maximum(_dot3(z, w1_ref[...]), 0.0)
    z = jnp.maximum(_dot3(z, w2_ref[...]), 0.0)
    o_ref[...] = z


def _mlp(h, agg2, w1, w2):
    blk = 1000
    return pl.pallas_call(
        _mlp_body,
        grid=(N_NODES // blk,),
        in_specs=[
            pl.BlockSpec((blk, FEAT), lambda i: (i, 0)),
            pl.BlockSpec((1, blk, FEAT), lambda i: (0, i, 0)),
            pl.BlockSpec((1, blk, FEAT), lambda i: (1, i, 0)),
            pl.BlockSpec((FEAT, FEAT), lambda i: (0, 0)),
            pl.BlockSpec((FEAT, FEAT), lambda i: (0, 0)),
        ],
        out_specs=pl.BlockSpec((blk, FEAT), lambda i: (i, 0)),
        out_shape=jax.ShapeDtypeStruct((N_NODES, FEAT), jnp.float32),
    )(h, agg2, agg2, w1, w2)


def _pool_body(h_ref, b_ref, ow_ref, ob_ref, out_ref, off_sm, maxs_vm):
    batv = b_ref[...]

    def off_body(g, carry):
        off_sm[g] = jnp.sum((batv < g).astype(jnp.int32))
        return carry

    lax.fori_loop(0, N_GRAPHS + 1, off_body, 0)

    gid = lax.broadcasted_iota(jnp.int32, (N_GRAPHS, N_NODES), 0)
    onehot = (gid == batv).astype(jnp.float32)
    sums = _dot3(onehot, h_ref[...])
    counts = jnp.sum(onehot, axis=1, keepdims=True)

    RB = 256

    def g_body(g, carry):
        start = off_sm[g]
        end = off_sm[g + 1]
        nb = (end - start + RB - 1) // RB

        def k_body(k, acc):
            st = start + k * RB
            stc = jnp.minimum(st, N_NODES - RB)
            rows = h_ref[pl.ds(stc, RB), :]
            ridx = stc + lax.broadcasted_iota(jnp.int32, (RB, 1), 0)
            m = (ridx >= start) & (ridx < end)
            blkmax = jnp.max(jnp.where(m, rows, -jnp.inf), axis=0, keepdims=True)
            return jnp.maximum(acc, blkmax)

        acc = lax.fori_loop(0, nb, k_body,
                            jnp.full((1, FEAT), -jnp.inf, jnp.float32))
        maxs_vm[pl.ds(g, 1), :] = acc
        return carry

    lax.fori_loop(0, N_GRAPHS, g_body, 0)

    maxs = jnp.where(counts > 0, maxs_vm[...], 0.0)
    means = sums / jnp.maximum(counts, 1.0)
    pooled = jnp.concatenate([maxs, means, sums], axis=1)
    out_ref[...] = _dot3(pooled, ow_ref[...]) + ob_ref[...]


def _pool(h, batch2d, out_w, out_b2d):
    return pl.pallas_call(
        _pool_body,
        in_specs=[
            pl.BlockSpec((N_NODES, FEAT), lambda: (0, 0)),
            pl.BlockSpec((1, N_NODES), lambda: (0, 0)),
            pl.BlockSpec((3 * FEAT, FEAT), lambda: (0, 0)),
            pl.BlockSpec((1, FEAT), lambda: (0, 0)),
        ],
        out_specs=pl.BlockSpec((N_GRAPHS, FEAT), lambda: (0, 0)),
        out_shape=jax.ShapeDtypeStruct((N_GRAPHS, FEAT), jnp.float32),
        scratch_shapes=[
            pltpu.SMEM((N_GRAPHS + 1,), jnp.int32),
            pltpu.VMEM((N_GRAPHS, FEAT), jnp.float32),
        ],
    )(h, batch2d, out_w, out_b2d)


def kernel(x, edge_index, batch_index, w1_0, w2_0, w1_1, w2_1, w1_2, w2_2,
           out_w, out_b):
    src, dst = edge_index[0], edge_index[1]
    pad_n = E_PAD - N_EDGES
    pad_i = jnp.arange(pad_n, dtype=jnp.int32)
    src_p = jnp.concatenate([src, pad_i % N_NODES])
    dst_p = jnp.concatenate([dst, N_NODES + (pad_i % (N_PAD - N_NODES))])
    src_r = src_p.reshape(NC, NS, CH_PER_TILE, CHUNK)
    dst_r = dst_p.reshape(NC, NS, CH_PER_TILE, CHUNK)
    zeros = jnp.zeros((STRIPE, FEAT), jnp.float32)
    batch2d = batch_index.reshape(1, N_NODES)
    out_b2d = out_b.reshape(1, FEAT)

    h = x
    for (w1, w2) in ((w1_0, w2_0), (w1_1, w2_1), (w1_2, w2_2)):
        agg2 = _sc_aggregate(h, src_r, dst_r, zeros)
        h = _mlp(h, agg2, w1, w2)
    return _pool(h, batch2d, out_w, out_b2d)
